# Initial kernel scaffold; baseline (speedup 1.0000x reference)
import functools

import jax
import jax.numpy as jnp
from jax.experimental import pallas as pl
from jax.experimental.pallas import tpu as pltpu

B = 16384
INPUT_DIM = 512
H1 = 512
H2 = 256
LATENT = 32
K = 8192
NCLS = 10
BETA = 0.25

MB_A = 2048
MB_B = 1024
KC = 2048
MB_C = 2048


def _enc_body(x_ref, w1_ref, b1_ref, w2_ref, b2_ref, w3_ref, b3_ref, ze_ref):
    h = jnp.maximum(
        jnp.dot(x_ref[...], w1_ref[...], preferred_element_type=jnp.float32)
        + b1_ref[...], 0.0)
    h = jnp.maximum(
        jnp.dot(h, w2_ref[...], preferred_element_type=jnp.float32)
        + b2_ref[...], 0.0)
    ze_ref[...] = (
        jnp.dot(h, w3_ref[...], preferred_element_type=jnp.float32)
        + b3_ref[...])


def _quant_body(ze_ref, zn_ref, en_ref, cbt_ref, idx_ref):
    ze = ze_ref[...]
    zn = zn_ref[...]
    run_m = jnp.full((MB_B, 1), jnp.inf, dtype=jnp.float32)
    run_i = jnp.zeros((MB_B, 1), dtype=jnp.int32)
    for c in range(K // KC):
        mm = jnp.dot(ze, cbt_ref[:, pl.ds(c * KC, KC)],
                     preferred_element_type=jnp.float32)
        d = (zn + en_ref[:, pl.ds(c * KC, KC)]) - 2.0 * mm
        m = jnp.min(d, axis=1, keepdims=True)
        col = jax.lax.broadcasted_iota(jnp.int32, (MB_B, KC), 1) + c * KC
        i = jnp.min(jnp.where(d == m, col, jnp.int32(2**30)),
                    axis=1, keepdims=True)
        upd = m < run_m
        run_i = jnp.where(upd, i, run_i)
        run_m = jnp.where(upd, m, run_m)
    idx_ref[...] = run_i


def _dec_body(zq_ref, ze_ref, counts_ref, w1_ref, b1_ref, w2_ref, b2_ref,
              w3_ref, b3_ref, xr_ref, sq_ref, us_ref):
    i = pl.program_id(0)
    zq = zq_ref[...]
    d = jnp.maximum(
        jnp.dot(zq, w1_ref[...], preferred_element_type=jnp.float32)
        + b1_ref[...], 0.0)
    d = jnp.maximum(
        jnp.dot(d, w2_ref[...], preferred_element_type=jnp.float32)
        + b2_ref[...], 0.0)
    xr_ref[...] = (
        jnp.dot(d, w3_ref[...], preferred_element_type=jnp.float32)
        + b3_ref[...])
    sq = jnp.sum((zq - ze_ref[...]) ** 2)

    @pl.when(i == 0)
    def _():
        sq_ref[0, 0] = 0.0
        counts = counts_ref[...]
        total = jnp.sum(counts)
        probs = counts / (total + 1e-08)
        valid = probs > 0
        n_valid = jnp.sum(valid.astype(jnp.float32))
        safe_p = jnp.where(valid, probs, 1.0)
        us_ref[0, 0] = jnp.sum(
            jnp.where(valid, probs * jnp.log(safe_p * n_valid), 0.0))

    sq_ref[0, 0] += sq


def _encoder(x, w1, b1, w2, b2, w3, b3):
    grid = B // MB_A
    return pl.pallas_call(
        _enc_body,
        grid=(grid,),
        in_specs=[
            pl.BlockSpec((MB_A, INPUT_DIM), lambda i: (i, 0)),
            pl.BlockSpec((INPUT_DIM, H1), lambda i: (0, 0)),
            pl.BlockSpec((1, H1), lambda i: (0, 0)),
            pl.BlockSpec((H1, H2), lambda i: (0, 0)),
            pl.BlockSpec((1, H2), lambda i: (0, 0)),
            pl.BlockSpec((H2, LATENT), lambda i: (0, 0)),
            pl.BlockSpec((1, LATENT), lambda i: (0, 0)),
        ],
        out_specs=pl.BlockSpec((MB_A, LATENT), lambda i: (i, 0)),
        out_shape=jax.ShapeDtypeStruct((B, LATENT), jnp.float32),
    )(x, w1, b1, w2, b2, w3, b3)


def _quantize(ze, zn, en, cbt):
    grid = B // MB_B
    return pl.pallas_call(
        _quant_body,
        grid=(grid,),
        in_specs=[
            pl.BlockSpec((MB_B, LATENT), lambda i: (i, 0)),
            pl.BlockSpec((MB_B, 1), lambda i: (i, 0)),
            pl.BlockSpec((1, K), lambda i: (0, 0)),
            pl.BlockSpec((LATENT, K), lambda i: (0, 0)),
        ],
        out_specs=pl.BlockSpec((MB_B, 1), lambda i: (i, 0)),
        out_shape=jax.ShapeDtypeStruct((B, 1), jnp.int32),
    )(ze, zn, en, cbt)


def _decoder(zq, ze, counts, w1, b1, w2, b2, w3, b3):
    grid = B // MB_C
    return pl.pallas_call(
        _dec_body,
        grid=(grid,),
        in_specs=[
            pl.BlockSpec((MB_C, LATENT), lambda i: (i, 0)),
            pl.BlockSpec((MB_C, LATENT), lambda i: (i, 0)),
            pl.BlockSpec((1, K), lambda i: (0, 0)),
            pl.BlockSpec((LATENT, H2), lambda i: (0, 0)),
            pl.BlockSpec((1, H2), lambda i: (0, 0)),
            pl.BlockSpec((H2, H1), lambda i: (0, 0)),
            pl.BlockSpec((1, H1), lambda i: (0, 0)),
            pl.BlockSpec((H1, INPUT_DIM), lambda i: (0, 0)),
            pl.BlockSpec((1, INPUT_DIM), lambda i: (0, 0)),
        ],
        out_specs=[
            pl.BlockSpec((MB_C, INPUT_DIM), lambda i: (i, 0)),
            pl.BlockSpec((1, 1), lambda i: (0, 0)),
            pl.BlockSpec((1, 1), lambda i: (0, 0)),
        ],
        out_shape=[
            jax.ShapeDtypeStruct((B, INPUT_DIM), jnp.float32),
            jax.ShapeDtypeStruct((1, 1), jnp.float32),
            jax.ShapeDtypeStruct((1, 1), jnp.float32),
        ],
    )(zq, ze, counts, w1, b1, w2, b2, w3, b3)


def kernel(x, y, enc_W1, enc_b1, enc_W2, enc_b2, enc_W3, enc_b3,
           dec_W1, dec_b1, dec_W2, dec_b2, dec_W3, dec_b3, codebook):
    del y
    z_e = _encoder(x, enc_W1, enc_b1.reshape(1, H1), enc_W2,
                   enc_b2.reshape(1, H2), enc_W3, enc_b3.reshape(1, LATENT))
    z_norm2 = jnp.sum(z_e ** 2, axis=1, keepdims=True)
    e_norm2 = jnp.sum(codebook ** 2, axis=1)[None, :]
    cbt = codebook.T
    code_idx = _quantize(z_e, z_norm2, e_norm2, cbt).reshape(B)

    z_q = jnp.take(codebook, code_idx, axis=0)
    counts = jnp.bincount(code_idx, length=K).astype(jnp.float32)[None, :]

    x_rec, sq, usage_loss = _decoder(
        z_q, z_e, counts, dec_W1, dec_b1.reshape(1, H2), dec_W2,
        dec_b2.reshape(1, H1), dec_W3, dec_b3.reshape(1, INPUT_DIM))
    m = sq[0, 0] / jnp.float32(B * LATENT)
    vq_loss = m + BETA * m
    return (x_rec, vq_loss, usage_loss[0, 0], code_idx)


# speedup vs baseline: 1.1132x; 1.1132x over previous
import functools

import jax
import jax.numpy as jnp
from jax.experimental import pallas as pl
from jax.experimental.pallas import tpu as pltpu

B = 16384
INPUT_DIM = 512
H1 = 512
H2 = 256
LATENT = 32
K = 8192
NCLS = 10
BETA = 0.25

MB_A = 2048
MB_B = 512
KC = 4096
MB_C = 2048


def _bf16_dot(a, b):
    kdim = a.shape[1]
    acc = None
    for k0 in range(0, kdim, 256):
        p = jnp.dot(a[:, k0:k0 + 256].astype(jnp.bfloat16),
                    b[k0:k0 + 256, :].astype(jnp.bfloat16),
                    preferred_element_type=jnp.float32)
        acc = p if acc is None else acc + p
    return acc


def _enc_body(x_ref, w1_ref, b1_ref, w2_ref, b2_ref, w3_ref, b3_ref, ze_ref):
    h = jnp.maximum(_bf16_dot(x_ref[...], w1_ref[...]) + b1_ref[...], 0.0)
    h = jnp.maximum(_bf16_dot(h, w2_ref[...]) + b2_ref[...], 0.0)
    ze_ref[...] = _bf16_dot(h, w3_ref[...]) + b3_ref[...]


def _quant_body(ze_ref, zn_ref, en_ref, cbt_ref, idx_ref):
    ze = ze_ref[...]
    zn = zn_ref[...]
    run_m = jnp.full((MB_B, 1), jnp.inf, dtype=jnp.float32)
    run_i = jnp.zeros((MB_B, 1), dtype=jnp.int32)
    for c in range(K // KC):
        mm = jnp.dot(ze, cbt_ref[:, pl.ds(c * KC, KC)],
                     preferred_element_type=jnp.float32)
        d = (zn + en_ref[:, pl.ds(c * KC, KC)]) - 2.0 * mm
        m = jnp.min(d, axis=1, keepdims=True)
        col = jax.lax.broadcasted_iota(jnp.int32, (MB_B, KC), 1) + c * KC
        i = jnp.min(jnp.where(d == m, col, jnp.int32(2**30)),
                    axis=1, keepdims=True)
        upd = m < run_m
        run_i = jnp.where(upd, i, run_i)
        run_m = jnp.where(
            upd, m.astype(jnp.bfloat16).astype(jnp.float32), run_m)
    idx_ref[...] = run_i


def _dec_body(zq_ref, ze_ref, counts_ref, w1_ref, b1_ref, w2_ref, b2_ref,
              w3_ref, b3_ref, xr_ref, sq_ref, us_ref):
    i = pl.program_id(0)
    zq = zq_ref[...]
    d = jnp.maximum(_bf16_dot(zq, w1_ref[...]) + b1_ref[...], 0.0)
    d = jnp.maximum(_bf16_dot(d, w2_ref[...]) + b2_ref[...], 0.0)
    xr_ref[...] = _bf16_dot(d, w3_ref[...]) + b3_ref[...]
    sq = jnp.sum((zq - ze_ref[...]) ** 2, axis=(0, 1), keepdims=True)

    @pl.when(i == 0)
    def _():
        sq_ref[...] = jnp.zeros((1, 1), jnp.float32)
        counts = counts_ref[...]
        total = jnp.sum(counts, axis=1, keepdims=True)
        probs = counts / (total + 1e-08)
        valid = probs > 0
        n_valid = jnp.sum(valid.astype(jnp.float32), axis=1, keepdims=True)
        safe_p = jnp.where(valid, probs, 1.0)
        us_ref[...] = jnp.sum(
            jnp.where(valid, probs * jnp.log(safe_p * n_valid), 0.0),
            axis=1, keepdims=True)

    sq_ref[...] += sq


def _encoder(x, w1, b1, w2, b2, w3, b3):
    grid = B // MB_A
    return pl.pallas_call(
        _enc_body,
        grid=(grid,),
        in_specs=[
            pl.BlockSpec((MB_A, INPUT_DIM), lambda i: (i, 0)),
            pl.BlockSpec((INPUT_DIM, H1), lambda i: (0, 0)),
            pl.BlockSpec((1, H1), lambda i: (0, 0)),
            pl.BlockSpec((H1, H2), lambda i: (0, 0)),
            pl.BlockSpec((1, H2), lambda i: (0, 0)),
            pl.BlockSpec((H2, LATENT), lambda i: (0, 0)),
            pl.BlockSpec((1, LATENT), lambda i: (0, 0)),
        ],
        out_specs=pl.BlockSpec((MB_A, LATENT), lambda i: (i, 0)),
        out_shape=jax.ShapeDtypeStruct((B, LATENT), jnp.float32),
    )(x, w1, b1, w2, b2, w3, b3)


def _quantize(ze, zn, en, cbt):
    grid = B // MB_B
    return pl.pallas_call(
        _quant_body,
        grid=(grid,),
        in_specs=[
            pl.BlockSpec((MB_B, LATENT), lambda i: (i, 0)),
            pl.BlockSpec((MB_B, 1), lambda i: (i, 0)),
            pl.BlockSpec((1, K), lambda i: (0, 0)),
            pl.BlockSpec((LATENT, K), lambda i: (0, 0)),
        ],
        out_specs=pl.BlockSpec((MB_B, 1), lambda i: (i, 0)),
        out_shape=jax.ShapeDtypeStruct((B, 1), jnp.int32),
    )(ze, zn, en, cbt)


def _decoder(zq, ze, counts, w1, b1, w2, b2, w3, b3):
    grid = B // MB_C
    return pl.pallas_call(
        _dec_body,
        grid=(grid,),
        in_specs=[
            pl.BlockSpec((MB_C, LATENT), lambda i: (i, 0)),
            pl.BlockSpec((MB_C, LATENT), lambda i: (i, 0)),
            pl.BlockSpec((1, K), lambda i: (0, 0)),
            pl.BlockSpec((LATENT, H2), lambda i: (0, 0)),
            pl.BlockSpec((1, H2), lambda i: (0, 0)),
            pl.BlockSpec((H2, H1), lambda i: (0, 0)),
            pl.BlockSpec((1, H1), lambda i: (0, 0)),
            pl.BlockSpec((H1, INPUT_DIM), lambda i: (0, 0)),
            pl.BlockSpec((1, INPUT_DIM), lambda i: (0, 0)),
        ],
        out_specs=[
            pl.BlockSpec((MB_C, INPUT_DIM), lambda i: (i, 0)),
            pl.BlockSpec((1, 1), lambda i: (0, 0)),
            pl.BlockSpec((1, 1), lambda i: (0, 0)),
        ],
        out_shape=[
            jax.ShapeDtypeStruct((B, INPUT_DIM), jnp.float32),
            jax.ShapeDtypeStruct((1, 1), jnp.float32),
            jax.ShapeDtypeStruct((1, 1), jnp.float32),
        ],
    )(zq, ze, counts, w1, b1, w2, b2, w3, b3)


def kernel(x, y, enc_W1, enc_b1, enc_W2, enc_b2, enc_W3, enc_b3,
           dec_W1, dec_b1, dec_W2, dec_b2, dec_W3, dec_b3, codebook):
    del y
    z_e = _encoder(x, enc_W1, enc_b1.reshape(1, H1), enc_W2,
                   enc_b2.reshape(1, H2), enc_W3, enc_b3.reshape(1, LATENT))
    z_norm2 = jnp.sum(z_e ** 2, axis=1, keepdims=True)
    e_norm2 = jnp.sum(codebook ** 2, axis=1)[None, :]
    cbt = codebook.T
    code_idx = _quantize(z_e, z_norm2, e_norm2, cbt).reshape(B)

    z_q = jnp.take(codebook, code_idx, axis=0)
    counts = jnp.bincount(code_idx, length=K).astype(jnp.float32)[None, :]

    x_rec, sq, usage_loss = _decoder(
        z_q, z_e, counts, dec_W1, dec_b1.reshape(1, H2), dec_W2,
        dec_b2.reshape(1, H1), dec_W3, dec_b3.reshape(1, INPUT_DIM))
    m = sq[0, 0] / jnp.float32(B * LATENT)
    vq_loss = m + BETA * m
    return (x_rec, vq_loss, usage_loss[0, 0], code_idx)


# speedup vs baseline: 1.1319x; 1.0168x over previous
import functools

import jax
import jax.numpy as jnp
from jax.experimental import pallas as pl
from jax.experimental.pallas import tpu as pltpu
from jax.experimental.pallas import tpu_sc as plsc

B = 16384
INPUT_DIM = 512
H1 = 512
H2 = 256
LATENT = 32
K = 8192
NCLS = 10
BETA = 0.25

MB_A = 2048
MB_B = 512
KC = 4096
MB_C = 2048


def _bf16_dot(a, b):
    kdim = a.shape[1]
    acc = None
    for k0 in range(0, kdim, 256):
        p = jnp.dot(a[:, k0:k0 + 256].astype(jnp.bfloat16),
                    b[k0:k0 + 256, :].astype(jnp.bfloat16),
                    preferred_element_type=jnp.float32)
        acc = p if acc is None else acc + p
    return acc


def _enc_body(x_ref, w1_ref, b1_ref, w2_ref, b2_ref, w3_ref, b3_ref, ze_ref):
    h = jnp.maximum(_bf16_dot(x_ref[...], w1_ref[...]) + b1_ref[...], 0.0)
    h = jnp.maximum(_bf16_dot(h, w2_ref[...]) + b2_ref[...], 0.0)
    ze_ref[...] = _bf16_dot(h, w3_ref[...]) + b3_ref[...]


def _quant_body(ze_ref, zn_ref, en_ref, cbt_ref, idx_ref):
    ze = ze_ref[...]
    zn = zn_ref[...]
    run_m = jnp.full((MB_B, 1), jnp.inf, dtype=jnp.float32)
    run_i = jnp.zeros((MB_B, 1), dtype=jnp.int32)
    for c in range(K // KC):
        mm = jnp.dot(ze, cbt_ref[:, pl.ds(c * KC, KC)],
                     preferred_element_type=jnp.float32)
        d = (zn + en_ref[:, pl.ds(c * KC, KC)]) - 2.0 * mm
        m = jnp.min(d, axis=1, keepdims=True)
        col = jax.lax.broadcasted_iota(jnp.int32, (MB_B, KC), 1) + c * KC
        i = jnp.min(jnp.where(d == m, col, jnp.int32(2**30)),
                    axis=1, keepdims=True)
        upd = m < run_m
        run_i = jnp.where(upd, i, run_i)
        run_m = jnp.where(
            upd, m.astype(jnp.bfloat16).astype(jnp.float32), run_m)
    idx_ref[...] = run_i


def _dec_body(zq_ref, ze_ref, counts_ref, w1_ref, b1_ref, w2_ref, b2_ref,
              w3_ref, b3_ref, xr_ref, sq_ref, us_ref):
    i = pl.program_id(0)
    zq = zq_ref[...]
    d = jnp.maximum(_bf16_dot(zq, w1_ref[...]) + b1_ref[...], 0.0)
    d = jnp.maximum(_bf16_dot(d, w2_ref[...]) + b2_ref[...], 0.0)
    xr_ref[...] = _bf16_dot(d, w3_ref[...]) + b3_ref[...]
    sq = jnp.sum((zq - ze_ref[...]) ** 2, axis=(0, 1), keepdims=True)

    @pl.when(i == 0)
    def _():
        sq_ref[...] = jnp.zeros((1, 1), jnp.float32)
        counts = counts_ref[...]
        total = jnp.sum(counts, axis=1, keepdims=True)
        probs = counts / (total + 1e-08)
        valid = probs > 0
        n_valid = jnp.sum(valid.astype(jnp.float32), axis=1, keepdims=True)
        safe_p = jnp.where(valid, probs, 1.0)
        us_ref[...] = jnp.sum(
            jnp.where(valid, probs * jnp.log(safe_p * n_valid), 0.0),
            axis=1, keepdims=True)

    sq_ref[...] += sq


def _encoder(x, w1, b1, w2, b2, w3, b3):
    grid = B // MB_A
    return pl.pallas_call(
        _enc_body,
        grid=(grid,),
        in_specs=[
            pl.BlockSpec((MB_A, INPUT_DIM), lambda i: (i, 0)),
            pl.BlockSpec((INPUT_DIM, H1), lambda i: (0, 0)),
            pl.BlockSpec((1, H1), lambda i: (0, 0)),
            pl.BlockSpec((H1, H2), lambda i: (0, 0)),
            pl.BlockSpec((1, H2), lambda i: (0, 0)),
            pl.BlockSpec((H2, LATENT), lambda i: (0, 0)),
            pl.BlockSpec((1, LATENT), lambda i: (0, 0)),
        ],
        out_specs=pl.BlockSpec((MB_A, LATENT), lambda i: (i, 0)),
        out_shape=jax.ShapeDtypeStruct((B, LATENT), jnp.float32),
    )(x, w1, b1, w2, b2, w3, b3)


def _quantize(ze, zn, en, cbt):
    grid = B // MB_B
    return pl.pallas_call(
        _quant_body,
        grid=(grid,),
        in_specs=[
            pl.BlockSpec((MB_B, LATENT), lambda i: (i, 0)),
            pl.BlockSpec((MB_B, 1), lambda i: (i, 0)),
            pl.BlockSpec((1, K), lambda i: (0, 0)),
            pl.BlockSpec((LATENT, K), lambda i: (0, 0)),
        ],
        out_specs=pl.BlockSpec((MB_B, 1), lambda i: (i, 0)),
        out_shape=jax.ShapeDtypeStruct((B, 1), jnp.int32),
    )(ze, zn, en, cbt)


def _decoder(zq, ze, counts, w1, b1, w2, b2, w3, b3):
    grid = B // MB_C
    return pl.pallas_call(
        _dec_body,
        grid=(grid,),
        in_specs=[
            pl.BlockSpec((MB_C, LATENT), lambda i: (i, 0)),
            pl.BlockSpec((MB_C, LATENT), lambda i: (i, 0)),
            pl.BlockSpec((1, K), lambda i: (0, 0)),
            pl.BlockSpec((LATENT, H2), lambda i: (0, 0)),
            pl.BlockSpec((1, H2), lambda i: (0, 0)),
            pl.BlockSpec((H2, H1), lambda i: (0, 0)),
            pl.BlockSpec((1, H1), lambda i: (0, 0)),
            pl.BlockSpec((H1, INPUT_DIM), lambda i: (0, 0)),
            pl.BlockSpec((1, INPUT_DIM), lambda i: (0, 0)),
        ],
        out_specs=[
            pl.BlockSpec((MB_C, INPUT_DIM), lambda i: (i, 0)),
            pl.BlockSpec((1, 1), lambda i: (0, 0)),
            pl.BlockSpec((1, 1), lambda i: (0, 0)),
        ],
        out_shape=[
            jax.ShapeDtypeStruct((B, INPUT_DIM), jnp.float32),
            jax.ShapeDtypeStruct((1, 1), jnp.float32),
            jax.ShapeDtypeStruct((1, 1), jnp.float32),
        ],
    )(zq, ze, counts, w1, b1, w2, b2, w3, b3)


_SC_NW = 32
_SC_BPW = B // _SC_NW


def _sc_gather(codebook, idx):
    mesh = plsc.VectorSubcoreMesh(core_axis_name="c", subcore_axis_name="s")

    @functools.partial(
        pl.kernel, mesh=mesh,
        out_type=jax.ShapeDtypeStruct((B, 128), jnp.float32),
        scratch_types=[
            pltpu.VMEM((_SC_BPW,), jnp.int32),
            pltpu.VMEM((_SC_BPW, 128), jnp.float32),
            pltpu.SemaphoreType.DMA,
        ],
    )
    def k(cb_hbm, idx_hbm, out_hbm, idx_v, rows_v, sem):
        wid = jax.lax.axis_index("s") * 2 + jax.lax.axis_index("c")
        base = wid * _SC_BPW
        pltpu.sync_copy(idx_hbm.at[pl.ds(base, _SC_BPW)], idx_v)
        pltpu.async_copy(cb_hbm.at[idx_v], rows_v, sem).wait()
        pltpu.sync_copy(rows_v, out_hbm.at[pl.ds(base, _SC_BPW)])

    return k(codebook, idx)


def kernel(x, y, enc_W1, enc_b1, enc_W2, enc_b2, enc_W3, enc_b3,
           dec_W1, dec_b1, dec_W2, dec_b2, dec_W3, dec_b3, codebook):
    del y
    z_e = _encoder(x, enc_W1, enc_b1.reshape(1, H1), enc_W2,
                   enc_b2.reshape(1, H2), enc_W3, enc_b3.reshape(1, LATENT))
    z_norm2 = jnp.sum(z_e ** 2, axis=1, keepdims=True)
    e_norm2 = jnp.sum(codebook ** 2, axis=1)[None, :]
    cbt = codebook.T
    code_idx = _quantize(z_e, z_norm2, e_norm2, cbt).reshape(B)

    cb_pad = jnp.pad(codebook, ((0, 0), (0, 128 - LATENT)))
    z_q = _sc_gather(cb_pad, code_idx)[:, :LATENT]
    counts = jnp.bincount(code_idx, length=K).astype(jnp.float32)[None, :]

    x_rec, sq, usage_loss = _decoder(
        z_q, z_e, counts, dec_W1, dec_b1.reshape(1, H2), dec_W2,
        dec_b2.reshape(1, H1), dec_W3, dec_b3.reshape(1, INPUT_DIM))
    m = sq[0, 0] / jnp.float32(B * LATENT)
    vq_loss = m + BETA * m
    return (x_rec, vq_loss, usage_loss[0, 0], code_idx)
